# baseline (speedup 1.0000x reference)
import jax
import jax.numpy as jnp
from jax.experimental import pallas as pl

B = 1024
D = 64
NE = 100000
K = 32
L = 4
NC = 16


def _copy_body(x_ref, o_ref):
    o_ref[...] = x_ref[...]


def kernel(graph_feature, evidence_emb, emb_label, label_emb):
    score = jnp.dot(graph_feature, evidence_emb.T)
    kvalue, kind = jax.lax.top_k(score, K)
    kvalue = jnp.sqrt(kvalue)
    kvalue = jax.nn.softmax(kvalue, axis=-1)
    res = None
    for i in range(L):
        klabel = jnp.take(emb_label[i], kind)
        klabel_emb = jnp.take(label_emb[i], klabel, axis=0)
        weighted = klabel_emb * kvalue[..., None]
        r = weighted.sum(axis=1)
        res = r if res is None else res + r
    res = pl.pallas_call(
        _copy_body,
        out_shape=jax.ShapeDtypeStruct((B, D), jnp.float32),
    )(res)
    return res


# speedup vs baseline: 2.2595x; 2.2595x over previous
import functools

import jax
import jax.numpy as jnp
from jax import lax
from jax.experimental import pallas as pl
from jax.experimental.pallas import tpu as pltpu
from jax.experimental.pallas import tpu_sc as plsc

B = 1024
D = 64
NE = 100000
K = 32
L = 4
NC = 16

NE_PAD = 102400
CBLK = 512
NBLK = NE_PAD // CBLK

NW = 32
RW = B // NW
NCHUNK = 4
CS = NE_PAD // NCHUNK
GRP = 64
NG = CS // GRP
CAP = 256
COMPACT_AT = CAP - 64
SENT = -2.0
TAU0 = -0.5
MAXBITS = 0x42800000
IBIG = 0x7FFFFFFF


def _score_body(g_ref, e_ref, o_ref):
    j = pl.program_id(0)
    s = jnp.dot(g_ref[...], e_ref[...].T,
                preferred_element_type=jnp.float32)

    @pl.when(j < (NE // CBLK))
    def _():
        o_ref[...] = s

    @pl.when(j >= (NE // CBLK))
    def _():
        col = j * CBLK + jax.lax.broadcasted_iota(jnp.int32, (B, CBLK), 1)
        o_ref[...] = jnp.where(col < NE, s, -1.0)


def _scores(graph_feature, evidence_pad):
    return pl.pallas_call(
        _score_body,
        grid=(NBLK,),
        in_specs=[
            pl.BlockSpec((B, D), lambda j: (0, 0)),
            pl.BlockSpec((CBLK, D), lambda j: (j, 0)),
        ],
        out_specs=pl.BlockSpec((B, CBLK), lambda j: (0, j)),
        out_shape=jax.ShapeDtypeStruct((B, NE_PAD), jnp.float32),
    )(graph_feature, evidence_pad)


def _sc_topk(scores_flat, elab_flat):
    mesh = plsc.VectorSubcoreMesh(core_axis_name="c", subcore_axis_name="s",
                                  num_cores=2, num_subcores=16)

    @functools.partial(
        pl.kernel,
        out_type=(
            jax.ShapeDtypeStruct((B * K,), jnp.float32),
            jax.ShapeDtypeStruct((B * K,), jnp.int32),
            jax.ShapeDtypeStruct((L * B * K,), jnp.int32),
        ),
        mesh=mesh,
        scratch_types=[
            pltpu.VMEM((CS,), jnp.float32),
            pltpu.VMEM((CS,), jnp.float32),
            pltpu.VMEM((CAP,), jnp.float32),
            pltpu.VMEM((CAP,), jnp.int32),
            pltpu.VMEM((CAP,), jnp.int32),
            pltpu.VMEM((K,), jnp.int32),
            pltpu.VMEM((K,), jnp.int32),
            pltpu.SemaphoreType.DMA,
            pltpu.SemaphoreType.DMA,
            pltpu.SemaphoreType.DMA,
        ],
        compiler_params=pltpu.CompilerParams(needs_layout_passes=False),
    )
    def sc_kernel(s_hbm, elab_hbm, kval_hbm, kind_hbm, klab_hbm,
                  bufA, bufB, cval, cidx, tieb, gidx, glab,
                  semA, semB, semO):
        wid = lax.axis_index("s") * 2 + lax.axis_index("c")
        row0 = wid * RW
        iota = lax.iota(jnp.int32, 16)
        zero16 = jnp.zeros((16,), jnp.int32)

        def splat_f(x):
            return jnp.full((16,), x, jnp.float32)

        def splat_i(x):
            return jnp.full((16,), x, jnp.int32)

        def compact(tau_vec, cnt_vec):
            for j in range(CAP // 16):
                lanes = splat_i(j * 16) + iota
                v = cval[pl.ds(j * 16, 16)]
                cval[pl.ds(j * 16, 16)] = jnp.where(lanes < cnt_vec, v, SENT)

            def bis(_, lohi):
                lo, hi = lohi
                mid = lo + (hi - lo) // 2
                tv = jnp.full((16,), lax.bitcast_convert_type(mid, jnp.float32))
                cgt = zero16
                for j in range(CAP // 16):
                    m = cval[pl.ds(j * 16, 16)] > tv
                    cgt = cgt + plsc.all_reduce_population_count(m)
                blt = jnp.any(cgt < K)
                lo = jnp.where(blt, lo, mid + 1)
                hi = jnp.where(blt, mid, hi)
                return lo, hi

            lo, _ = lax.fori_loop(0, 31, bis, (jnp.int32(0), jnp.int32(MAXBITS)))
            tau = lax.bitcast_convert_type(lo, jnp.float32)
            tau_vec = splat_f(tau)

            cnt2 = zero16
            tcnt = zero16
            for j in range(CAP // 16):
                v = cval[pl.ds(j * 16, 16)]
                ix = cidx[pl.ds(j * 16, 16)]
                m = v > tau_vec
                mi = m.astype(jnp.int32)
                pos = cnt2 + plsc.cumsum(mi) - mi
                plsc.store_scatter(cval, [pos], v, mask=m)
                plsc.store_scatter(cidx, [pos], ix, mask=m)
                cnt2 = cnt2 + plsc.all_reduce_population_count(m)
                me = v == tau_vec
                mei = me.astype(jnp.int32)
                post = tcnt + plsc.cumsum(mei) - mei
                plsc.store_scatter(tieb, [post], ix, mask=me)
                tcnt = tcnt + plsc.all_reduce_population_count(me)

            for j in range(CAP // 16):
                lanes = splat_i(j * 16) + iota
                t = tieb[pl.ds(j * 16, 16)]
                tieb[pl.ds(j * 16, 16)] = jnp.where(lanes < tcnt,
                                                    t, jnp.int32(IBIG))
            ngt = jnp.max(cnt2)

            def fill(k, _):
                mn = tieb[pl.ds(0, 16)]
                for j in range(1, CAP // 16):
                    mn = jnp.minimum(mn, tieb[pl.ds(j * 16, 16)])
                mni = jnp.min(mn)
                p = splat_i(ngt + k)
                lane0 = iota == 0
                mnv = splat_i(mni)
                plsc.store_scatter(cval, [p], splat_f(tau), mask=lane0)
                plsc.store_scatter(cidx, [p], mnv, mask=lane0)
                for j in range(CAP // 16):
                    t = tieb[pl.ds(j * 16, 16)]
                    tieb[pl.ds(j * 16, 16)] = jnp.where(t == mnv,
                                                        jnp.int32(IBIG), t)
                return 0

            lax.fori_loop(0, K - ngt, fill, 0)
            return tau_vec, splat_i(K)

        def scan_chunk(buf, base, tau_vec, cnt_vec):
            def group(g, carry):
                tau_vec, cnt_vec = carry
                off = g * GRP
                vs = [buf[pl.ds(off + 16 * i, 16)] for i in range(GRP // 16)]
                gmax = vs[0]
                for v in vs[1:]:
                    gmax = jnp.maximum(gmax, v)
                pred = jnp.any(gmax > tau_vec)

                def do_insert(tv, cv):
                    for i, vi in enumerate(vs):
                        m = vi > tv
                        mi = m.astype(jnp.int32)
                        pos = cv + plsc.cumsum(mi) - mi
                        plsc.store_scatter(cval, [pos], vi, mask=m)
                        idxv = splat_i(base + off + 16 * i) + iota
                        plsc.store_scatter(cidx, [pos], idxv, mask=m)
                        cv = cv + plsc.all_reduce_population_count(m)
                    need = jnp.any(cv > COMPACT_AT)
                    return lax.cond(need, compact, lambda a, b: (a, b), tv, cv)

                return lax.cond(pred, do_insert,
                                lambda tv, cv: (tv, cv), tau_vec, cnt_vec)

            return lax.fori_loop(0, NG, group, (tau_vec, cnt_vec))

        def process_row(r, _):
            row = row0 + r
            rbase = row * NE_PAD
            tau_vec = splat_f(TAU0)
            cnt_vec = zero16
            cp0 = pltpu.async_copy(s_hbm.at[pl.ds(rbase + 0 * CS, CS)],
                                   bufA, semA)
            cp0.wait()
            cp1 = pltpu.async_copy(s_hbm.at[pl.ds(rbase + 1 * CS, CS)],
                                   bufB, semB)
            tau_vec, cnt_vec = scan_chunk(bufA, 0 * CS, tau_vec, cnt_vec)
            cp1.wait()
            cp2 = pltpu.async_copy(s_hbm.at[pl.ds(rbase + 2 * CS, CS)],
                                   bufA, semA)
            tau_vec, cnt_vec = scan_chunk(bufB, 1 * CS, tau_vec, cnt_vec)
            cp2.wait()
            cp3 = pltpu.async_copy(s_hbm.at[pl.ds(rbase + 3 * CS, CS)],
                                   bufB, semB)
            tau_vec, cnt_vec = scan_chunk(bufA, 2 * CS, tau_vec, cnt_vec)
            cp3.wait()
            tau_vec, cnt_vec = scan_chunk(bufB, 3 * CS, tau_vec, cnt_vec)

            compact(tau_vec, cnt_vec)

            pltpu.sync_copy(cval.at[pl.ds(0, K)],
                            kval_hbm.at[pl.ds(row * K, K)])
            pltpu.sync_copy(cidx.at[pl.ds(0, K)],
                            kind_hbm.at[pl.ds(row * K, K)])
            for i in range(L):
                gidx[pl.ds(0, 16)] = cidx[pl.ds(0, 16)] + splat_i(i * NE)
                gidx[pl.ds(16, 16)] = cidx[pl.ds(16, 16)] + splat_i(i * NE)
                pltpu.async_copy(elab_hbm.at[gidx], glab, semO).wait()
                pltpu.sync_copy(
                    glab, klab_hbm.at[pl.ds(i * B * K + row * K, K)])
            return 0

        lax.fori_loop(0, RW, process_row, 0)

    return sc_kernel(scores_flat, elab_flat)


def _combine_body(kv_ref, kl_ref, le_ref, o_ref):
    kv = kv_ref[...]
    w = jnp.sqrt(kv)
    w = w - jnp.max(w, axis=-1, keepdims=True)
    w = jnp.exp(w)
    w = w / jnp.sum(w, axis=-1, keepdims=True)
    kl = kl_ref[...]
    cls_iota = jax.lax.broadcasted_iota(jnp.int32, (B, K, NC), 2)
    ws = []
    for i in range(L):
        eq = (kl[i][:, :, None] == cls_iota).astype(jnp.float32)
        ws.append(jnp.sum(eq * w[:, :, None], axis=1))
    W = jnp.concatenate(ws, axis=-1)
    o_ref[...] = jnp.dot(W, le_ref[...], preferred_element_type=jnp.float32)


def _combine(kvalue, klabel_bkl, label_flat):
    return pl.pallas_call(
        _combine_body,
        out_shape=jax.ShapeDtypeStruct((B, D), jnp.float32),
    )(kvalue, klabel_bkl, label_flat)


def kernel(graph_feature, evidence_emb, emb_label, label_emb):
    evidence_pad = jnp.pad(evidence_emb, ((0, NE_PAD - NE), (0, 0)))
    scores = _scores(graph_feature, evidence_pad)
    scores_flat = scores.reshape(B * NE_PAD)
    elab_flat = emb_label.reshape(L * NE).astype(jnp.int32)
    kval_f, _, klab_f = _sc_topk(scores_flat, elab_flat)
    kvalue = kval_f.reshape(B, K)
    klabel = klab_f.reshape(L, B, K)
    label_flat = label_emb.reshape(L * NC, D)
    return _combine(kvalue, klabel, label_flat)


# speedup vs baseline: 5.9737x; 2.6438x over previous
import functools

import jax
import jax.numpy as jnp
from jax import lax
from jax.experimental import pallas as pl
from jax.experimental.pallas import tpu as pltpu
from jax.experimental.pallas import tpu_sc as plsc

B = 1024
D = 64
NE = 100000
K = 32
L = 4
NC = 16

NE_PAD = 102400
CBLK = 1024
NBLK = NE_PAD // CBLK
GSZ = 8
NGM = NE_PAD // GSZ

NW = 32
RW = B // NW
GRP = 64
NG = NGM // GRP
CAP = 256
COMPACT_AT = CAP - 64
SENT = -2.0
PADG = 12672
MAXBITS = 0x42800000
IBIG = 0x7FFFFFFF


def _score_body(g_ref, e_ref, o_ref, gm_ref):
    j = pl.program_id(0)
    s = jnp.dot(g_ref[...], e_ref[...].T,
                preferred_element_type=jnp.float32)

    @pl.when(j >= (NE // CBLK))
    def _():
        col = j * CBLK + jax.lax.broadcasted_iota(jnp.int32, (B, CBLK), 1)
        sp = jnp.where(col < NE, s, -1.0)
        o_ref[...] = sp
        gm_ref[...] = jnp.max(sp.reshape(B, GSZ, CBLK // GSZ), axis=1)

    @pl.when(j < (NE // CBLK))
    def _():
        o_ref[...] = s
        gm_ref[...] = jnp.max(s.reshape(B, GSZ, CBLK // GSZ), axis=1)


def _scores(graph_feature, evidence_pad):
    return pl.pallas_call(
        _score_body,
        grid=(NBLK,),
        in_specs=[
            pl.BlockSpec((B, D), lambda j: (0, 0)),
            pl.BlockSpec((CBLK, D), lambda j: (j, 0)),
        ],
        out_specs=[
            pl.BlockSpec((B, CBLK), lambda j: (0, j)),
            pl.BlockSpec((B, CBLK // GSZ), lambda j: (0, j)),
        ],
        out_shape=[
            jax.ShapeDtypeStruct((B, NE_PAD), jnp.float32),
            jax.ShapeDtypeStruct((B, NGM), jnp.float32),
        ],
    )(graph_feature, evidence_pad)


def _tau0_body(gm_ref, o_ref):
    gm = gm_ref[...]
    bt = gm.shape[0]
    bm = jnp.max(gm[:, :49 * 256].reshape(bt, 49, 256), axis=-1)
    t = jnp.min(bm, axis=-1)
    o_ref[...] = jnp.broadcast_to(t[:, None], (bt, 16))


def _tau0(gm):
    bt = B // 4
    return pl.pallas_call(
        _tau0_body,
        grid=(4,),
        in_specs=[pl.BlockSpec((bt, NGM), lambda i: (i, 0))],
        out_specs=pl.BlockSpec((bt, 16), lambda i: (i, 0)),
        out_shape=jax.ShapeDtypeStruct((B, 16), jnp.float32),
    )(gm)


def _sc_topk(scores_flat, gm_flat, tau_flat, elab_flat):
    mesh = plsc.VectorSubcoreMesh(core_axis_name="c", subcore_axis_name="s",
                                  num_cores=2, num_subcores=16)

    @functools.partial(
        pl.kernel,
        out_type=(
            jax.ShapeDtypeStruct((B * K,), jnp.float32),
            jax.ShapeDtypeStruct((B * K,), jnp.int32),
            jax.ShapeDtypeStruct((L * B * K,), jnp.int32),
        ),
        mesh=mesh,
        scratch_types=[
            pltpu.VMEM((NGM,), jnp.float32),
            pltpu.VMEM((RW * 16,), jnp.float32),
            pltpu.VMEM((CAP,), jnp.float32),
            pltpu.VMEM((CAP,), jnp.int32),
            pltpu.VMEM((CAP,), jnp.int32),
            pltpu.VMEM((K,), jnp.int32),
            pltpu.VMEM((K,), jnp.int32),
            pltpu.SemaphoreType.DMA,
            pltpu.SemaphoreType.DMA,
            pltpu.SemaphoreType.DMA,
        ],
        compiler_params=pltpu.CompilerParams(needs_layout_passes=False),
    )
    def sc_kernel(s_hbm, gm_hbm, tau_hbm, elab_hbm,
                  kval_hbm, kind_hbm, klab_hbm,
                  gmbuf, tbuf, cval, cidx, tieb, gidx, glab,
                  semA, semF, semO):
        wid = lax.axis_index("s") * 2 + lax.axis_index("c")
        row0 = wid * RW
        iota = lax.iota(jnp.int32, 16)
        zero16 = jnp.zeros((16,), jnp.int32)

        def splat_f(x):
            return jnp.full((16,), x, jnp.float32)

        def splat_i(x):
            return jnp.full((16,), x, jnp.int32)

        def compact(tau_vec, cnt_vec):
            for j in range(CAP // 16):
                lanes = splat_i(j * 16) + iota
                v = cval[pl.ds(j * 16, 16)]
                cval[pl.ds(j * 16, 16)] = jnp.where(lanes < cnt_vec, v, SENT)

            def bis(_, lohi):
                lo, hi = lohi
                mid = lo + (hi - lo) // 2
                tv = jnp.full((16,), lax.bitcast_convert_type(mid, jnp.float32))
                cgt = zero16
                for j in range(CAP // 16):
                    m = cval[pl.ds(j * 16, 16)] > tv
                    cgt = cgt + plsc.all_reduce_population_count(m)
                blt = jnp.any(cgt < K)
                lo = jnp.where(blt, lo, mid + 1)
                hi = jnp.where(blt, mid, hi)
                return lo, hi

            lo, _ = lax.fori_loop(0, 31, bis, (jnp.int32(0), jnp.int32(MAXBITS)))
            tau = lax.bitcast_convert_type(lo, jnp.float32)
            tau_vec = splat_f(tau)

            cnt2 = zero16
            tcnt = zero16
            for j in range(CAP // 16):
                v = cval[pl.ds(j * 16, 16)]
                ix = cidx[pl.ds(j * 16, 16)]
                m = v > tau_vec
                mi = m.astype(jnp.int32)
                pos = cnt2 + plsc.cumsum(mi) - mi
                plsc.store_scatter(cval, [pos], v, mask=m)
                plsc.store_scatter(cidx, [pos], ix, mask=m)
                cnt2 = cnt2 + plsc.all_reduce_population_count(m)
                me = v == tau_vec
                mei = me.astype(jnp.int32)
                post = tcnt + plsc.cumsum(mei) - mei
                plsc.store_scatter(tieb, [post], ix, mask=me)
                tcnt = tcnt + plsc.all_reduce_population_count(me)

            for j in range(CAP // 16):
                lanes = splat_i(j * 16) + iota
                t = tieb[pl.ds(j * 16, 16)]
                tieb[pl.ds(j * 16, 16)] = jnp.where(lanes < tcnt,
                                                    t, jnp.int32(IBIG))
            ngt = jnp.max(cnt2)

            def fill(k, _):
                mn = tieb[pl.ds(0, 16)]
                for j in range(1, CAP // 16):
                    mn = jnp.minimum(mn, tieb[pl.ds(j * 16, 16)])
                mni = jnp.min(mn)
                p = splat_i(ngt + k)
                lane0 = iota == 0
                mnv = splat_i(mni)
                plsc.store_scatter(cval, [p], splat_f(tau), mask=lane0)
                plsc.store_scatter(cidx, [p], mnv, mask=lane0)
                for j in range(CAP // 16):
                    t = tieb[pl.ds(j * 16, 16)]
                    tieb[pl.ds(j * 16, 16)] = jnp.where(t == mnv,
                                                        jnp.int32(IBIG), t)
                return 0

            lax.fori_loop(0, K - ngt, fill, 0)
            return tau_vec, splat_i(K)

        def scan_gm(tau_vec, cnt_vec):
            def group(g, carry):
                tau_vec, cnt_vec = carry
                off = g * GRP
                vs = [gmbuf[pl.ds(off + 16 * i, 16)] for i in range(GRP // 16)]
                gmax = vs[0]
                for v in vs[1:]:
                    gmax = jnp.maximum(gmax, v)
                pred = jnp.any(gmax >= tau_vec)

                def do_insert(tv, cv):
                    for i, vi in enumerate(vs):
                        m = vi >= tv
                        mi = m.astype(jnp.int32)
                        pos = cv + plsc.cumsum(mi) - mi
                        plsc.store_scatter(cval, [pos], vi, mask=m)
                        idxv = splat_i(off + 16 * i) + iota
                        plsc.store_scatter(cidx, [pos], idxv, mask=m)
                        cv = cv + plsc.all_reduce_population_count(m)
                    need = jnp.any(cv > COMPACT_AT)
                    return lax.cond(need, compact, lambda a, b: (a, b), tv, cv)

                return lax.cond(pred, do_insert,
                                lambda tv, cv: (tv, cv), tau_vec, cnt_vec)

            return lax.fori_loop(0, NG, group, (tau_vec, cnt_vec))

        def process_row(r, _):
            row = row0 + r
            rbase = row * NE_PAD
            cp = pltpu.async_copy(gm_hbm.at[pl.ds(row * NGM, NGM)],
                                  gmbuf, semA)
            cp.wait()
            tau_vec = tbuf[pl.ds(r * 16, 16)]
            cnt_vec = zero16
            tau_vec, cnt_vec = scan_gm(tau_vec, cnt_vec)
            tau_vec, cnt_vec = lax.cond(jnp.any(cnt_vec > K), compact,
                                        lambda a, b: (a, b), tau_vec, cnt_vec)
            for j in range(2):
                lanes = splat_i(j * 16) + iota
                ix = cidx[pl.ds(j * 16, 16)]
                cidx[pl.ds(j * 16, 16)] = jnp.where(lanes < cnt_vec,
                                                    ix, jnp.int32(PADG))
            g2 = [cidx[pl.ds(0, 16)], cidx[pl.ds(16, 16)]]
            cb = [((g >> 7) << 10) + (g & 127) for g in g2]
            cps = []
            for j in range(GSZ):
                for h in range(2):
                    gi = cb[h] + jnp.full((16,), rbase + 128 * j, jnp.int32)
                    cps.append(pltpu.async_copy(
                        s_hbm.at[gi],
                        cval.at[pl.ds(32 * j + 16 * h, 16)], semF))
                    cidx[pl.ds(32 * j + 16 * h, 16)] = (
                        cb[h] + splat_i(128 * j))
            for cp2 in cps:
                cp2.wait()
            compact(tau_vec, splat_i(CAP))

            pltpu.sync_copy(cval.at[pl.ds(0, K)],
                            kval_hbm.at[pl.ds(row * K, K)])
            pltpu.sync_copy(cidx.at[pl.ds(0, K)],
                            kind_hbm.at[pl.ds(row * K, K)])
            for i in range(L):
                gidx[pl.ds(0, 16)] = cidx[pl.ds(0, 16)] + splat_i(i * NE)
                gidx[pl.ds(16, 16)] = cidx[pl.ds(16, 16)] + splat_i(i * NE)
                pltpu.async_copy(elab_hbm.at[gidx], glab, semO).wait()
                pltpu.sync_copy(
                    glab, klab_hbm.at[pl.ds(i * B * K + row * K, K)])
            return 0

        pltpu.sync_copy(tau_hbm.at[pl.ds(row0 * 16, RW * 16)], tbuf)
        lax.fori_loop(0, RW, process_row, 0)

    return sc_kernel(scores_flat, gm_flat, tau_flat, elab_flat)


def _combine_body(kv_ref, kl_ref, le_ref, o_ref):
    kv = kv_ref[...]
    w = jnp.sqrt(kv)
    w = w - jnp.max(w, axis=-1, keepdims=True)
    w = jnp.exp(w)
    w = w / jnp.sum(w, axis=-1, keepdims=True)
    kl = kl_ref[...]
    cls_iota = jax.lax.broadcasted_iota(jnp.int32, (B, K, NC), 2)
    ws = []
    for i in range(L):
        eq = (kl[i][:, :, None] == cls_iota).astype(jnp.float32)
        ws.append(jnp.sum(eq * w[:, :, None], axis=1))
    W = jnp.concatenate(ws, axis=-1)
    o_ref[...] = jnp.dot(W, le_ref[...], preferred_element_type=jnp.float32)


def _combine(kvalue, klabel_bkl, label_flat):
    return pl.pallas_call(
        _combine_body,
        out_shape=jax.ShapeDtypeStruct((B, D), jnp.float32),
    )(kvalue, klabel_bkl, label_flat)


def kernel(graph_feature, evidence_emb, emb_label, label_emb):
    evidence_pad = jnp.pad(evidence_emb, ((0, NE_PAD - NE), (0, 0)))
    scores, gm = _scores(graph_feature, evidence_pad)
    tau0 = _tau0(gm)
    scores_flat = scores.reshape(B * NE_PAD)
    gm_flat = gm.reshape(B * NGM)
    tau_flat = tau0.reshape(B * 16)
    elab_flat = emb_label.reshape(L * NE).astype(jnp.int32)
    kval_f, _, klab_f = _sc_topk(scores_flat, gm_flat, tau_flat, elab_flat)
    kvalue = kval_f.reshape(B, K)
    klabel = klab_f.reshape(L, B, K)
    label_flat = label_emb.reshape(L * NC, D)
    return _combine(kvalue, klabel, label_flat)


# speedup vs baseline: 6.2271x; 1.0424x over previous
import functools

import jax
import jax.numpy as jnp
from jax import lax
from jax.experimental import pallas as pl
from jax.experimental.pallas import tpu as pltpu
from jax.experimental.pallas import tpu_sc as plsc

B = 1024
D = 64
NE = 100000
K = 32
L = 4
NC = 16

NE_PAD = 102400
CBLK = 1024
NBLK = NE_PAD // CBLK
GSZ = 8
NGM = NE_PAD // GSZ

NW = 32
RW = B // NW
GRP = 64
NG = NGM // GRP
CAP = 256
COMPACT_AT = CAP - 64
SENT = -2.0
PADG = 12672
MAXBITS = 0x42800000
IBIG = 0x7FFFFFFF


def _score_body(g_ref, e_ref, o_ref, gm_ref):
    j = pl.program_id(0)
    s = jnp.dot(g_ref[...], e_ref[...].T,
                preferred_element_type=jnp.float32)

    @pl.when(j >= (NE // CBLK))
    def _():
        col = j * CBLK + jax.lax.broadcasted_iota(jnp.int32, (B, CBLK), 1)
        sp = jnp.where(col < NE, s, -1.0)
        o_ref[...] = sp
        gm_ref[...] = jnp.max(sp.reshape(B, GSZ, CBLK // GSZ), axis=1)

    @pl.when(j < (NE // CBLK))
    def _():
        o_ref[...] = s
        gm_ref[...] = jnp.max(s.reshape(B, GSZ, CBLK // GSZ), axis=1)


def _scores(graph_feature, evidence_pad):
    return pl.pallas_call(
        _score_body,
        grid=(NBLK,),
        in_specs=[
            pl.BlockSpec((B, D), lambda j: (0, 0)),
            pl.BlockSpec((CBLK, D), lambda j: (j, 0)),
        ],
        out_specs=[
            pl.BlockSpec((B, CBLK), lambda j: (0, j)),
            pl.BlockSpec((B, CBLK // GSZ), lambda j: (0, j)),
        ],
        out_shape=[
            jax.ShapeDtypeStruct((B, NE_PAD), jnp.float32),
            jax.ShapeDtypeStruct((B, NGM), jnp.float32),
        ],
    )(graph_feature, evidence_pad)


def _tau0_body(gm_ref, o_ref):
    gm = gm_ref[...]
    bt = gm.shape[0]
    bm = jnp.max(gm[:, :49 * 256].reshape(bt, 49, 256), axis=-1)
    t = jnp.min(bm, axis=-1)
    o_ref[...] = jnp.broadcast_to(t[:, None], (bt, 16))


def _tau0(gm):
    bt = B // 4
    return pl.pallas_call(
        _tau0_body,
        grid=(4,),
        in_specs=[pl.BlockSpec((bt, NGM), lambda i: (i, 0))],
        out_specs=pl.BlockSpec((bt, 16), lambda i: (i, 0)),
        out_shape=jax.ShapeDtypeStruct((B, 16), jnp.float32),
    )(gm)


def _sc_topk(scores_flat, gm_flat, tau_flat, elab_flat):
    mesh = plsc.VectorSubcoreMesh(core_axis_name="c", subcore_axis_name="s",
                                  num_cores=2, num_subcores=16)

    @functools.partial(
        pl.kernel,
        out_type=(
            jax.ShapeDtypeStruct((B * K,), jnp.float32),
            jax.ShapeDtypeStruct((B * K,), jnp.int32),
            jax.ShapeDtypeStruct((L * B * K,), jnp.int32),
        ),
        mesh=mesh,
        scratch_types=[
            pltpu.VMEM((NGM,), jnp.float32),
            pltpu.VMEM((RW * 16,), jnp.float32),
            pltpu.VMEM((CAP,), jnp.float32),
            pltpu.VMEM((CAP,), jnp.int32),
            pltpu.VMEM((CAP,), jnp.int32),
            pltpu.VMEM((L * K,), jnp.int32),
            pltpu.VMEM((L * K,), jnp.int32),
            pltpu.SemaphoreType.DMA,
            pltpu.SemaphoreType.DMA,
            pltpu.SemaphoreType.DMA,
        ],
        compiler_params=pltpu.CompilerParams(needs_layout_passes=False),
    )
    def sc_kernel(s_hbm, gm_hbm, tau_hbm, elab_hbm,
                  kval_hbm, kind_hbm, klab_hbm,
                  gmbuf, tbuf, cval, cidx, tieb, gidx, glab,
                  semA, semF, semO):
        wid = lax.axis_index("s") * 2 + lax.axis_index("c")
        row0 = wid * RW
        iota = lax.iota(jnp.int32, 16)
        zero16 = jnp.zeros((16,), jnp.int32)

        def splat_f(x):
            return jnp.full((16,), x, jnp.float32)

        def splat_i(x):
            return jnp.full((16,), x, jnp.int32)

        def compact(tau_vec, cnt_vec):
            for j in range(CAP // 16):
                lanes = splat_i(j * 16) + iota
                v = cval[pl.ds(j * 16, 16)]
                cval[pl.ds(j * 16, 16)] = jnp.where(lanes < cnt_vec, v, SENT)

            def bis(_, lohi):
                lo, hi = lohi
                mid = lo + (hi - lo) // 2
                tv = jnp.full((16,), lax.bitcast_convert_type(mid, jnp.float32))
                cgt = zero16
                for j in range(CAP // 16):
                    m = cval[pl.ds(j * 16, 16)] > tv
                    cgt = cgt + plsc.all_reduce_population_count(m)
                blt = jnp.any(cgt < K)
                lo = jnp.where(blt, lo, mid + 1)
                hi = jnp.where(blt, mid, hi)
                return lo, hi

            lo, _ = lax.fori_loop(0, 31, bis, (jnp.int32(0), jnp.int32(MAXBITS)))
            tau = lax.bitcast_convert_type(lo, jnp.float32)
            tau_vec = splat_f(tau)

            cnt2 = zero16
            tcnt = zero16
            for j in range(CAP // 16):
                v = cval[pl.ds(j * 16, 16)]
                ix = cidx[pl.ds(j * 16, 16)]
                m = v > tau_vec
                mi = m.astype(jnp.int32)
                pos = cnt2 + plsc.cumsum(mi) - mi
                plsc.store_scatter(cval, [pos], v, mask=m)
                plsc.store_scatter(cidx, [pos], ix, mask=m)
                cnt2 = cnt2 + plsc.all_reduce_population_count(m)
                me = v == tau_vec
                mei = me.astype(jnp.int32)
                post = tcnt + plsc.cumsum(mei) - mei
                plsc.store_scatter(tieb, [post], ix, mask=me)
                tcnt = tcnt + plsc.all_reduce_population_count(me)

            for j in range(CAP // 16):
                lanes = splat_i(j * 16) + iota
                t = tieb[pl.ds(j * 16, 16)]
                tieb[pl.ds(j * 16, 16)] = jnp.where(lanes < tcnt,
                                                    t, jnp.int32(IBIG))
            ngt = jnp.max(cnt2)

            def fill(k, _):
                mn = tieb[pl.ds(0, 16)]
                for j in range(1, CAP // 16):
                    mn = jnp.minimum(mn, tieb[pl.ds(j * 16, 16)])
                mni = jnp.min(mn)
                p = splat_i(ngt + k)
                lane0 = iota == 0
                mnv = splat_i(mni)
                plsc.store_scatter(cval, [p], splat_f(tau), mask=lane0)
                plsc.store_scatter(cidx, [p], mnv, mask=lane0)
                for j in range(CAP // 16):
                    t = tieb[pl.ds(j * 16, 16)]
                    tieb[pl.ds(j * 16, 16)] = jnp.where(t == mnv,
                                                        jnp.int32(IBIG), t)
                return 0

            lax.fori_loop(0, K - ngt, fill, 0)
            return tau_vec, splat_i(K)

        def scan_gm(tau_vec, cnt_vec):
            def group(g, carry):
                tau_vec, cnt_vec = carry
                off = g * GRP
                vs = [gmbuf[pl.ds(off + 16 * i, 16)] for i in range(GRP // 16)]
                gmax = vs[0]
                for v in vs[1:]:
                    gmax = jnp.maximum(gmax, v)
                pred = jnp.any(gmax >= tau_vec)

                def do_insert(tv, cv):
                    for i, vi in enumerate(vs):
                        m = vi >= tv
                        mi = m.astype(jnp.int32)
                        pos = cv + plsc.cumsum(mi) - mi
                        plsc.store_scatter(cval, [pos], vi, mask=m)
                        idxv = splat_i(off + 16 * i) + iota
                        plsc.store_scatter(cidx, [pos], idxv, mask=m)
                        cv = cv + plsc.all_reduce_population_count(m)
                    need = jnp.any(cv > COMPACT_AT)
                    return lax.cond(need, compact, lambda a, b: (a, b), tv, cv)

                return lax.cond(pred, do_insert,
                                lambda tv, cv: (tv, cv), tau_vec, cnt_vec)

            return lax.fori_loop(0, NG, group, (tau_vec, cnt_vec))

        def process_row(r, _):
            row = row0 + r
            rbase = row * NE_PAD
            cp = pltpu.async_copy(gm_hbm.at[pl.ds(row * NGM, NGM)],
                                  gmbuf, semA)
            cp.wait()
            tau_vec = tbuf[pl.ds(r * 16, 16)]
            cnt_vec = zero16
            tau_vec, cnt_vec = scan_gm(tau_vec, cnt_vec)
            tau_vec, cnt_vec = lax.cond(jnp.any(cnt_vec > K), compact,
                                        lambda a, b: (a, b), tau_vec, cnt_vec)
            for j in range(2):
                lanes = splat_i(j * 16) + iota
                ix = cidx[pl.ds(j * 16, 16)]
                cidx[pl.ds(j * 16, 16)] = jnp.where(lanes < cnt_vec,
                                                    ix, jnp.int32(PADG))
            g2 = [cidx[pl.ds(0, 16)], cidx[pl.ds(16, 16)]]
            cb = [((g >> 7) << 10) + (g & 127) for g in g2]
            cps = []
            for j in range(GSZ):
                for h in range(2):
                    gi = cb[h] + jnp.full((16,), rbase + 128 * j, jnp.int32)
                    cps.append(pltpu.async_copy(
                        s_hbm.at[gi],
                        cval.at[pl.ds(32 * j + 16 * h, 16)], semF))
                    cidx[pl.ds(32 * j + 16 * h, 16)] = (
                        cb[h] + splat_i(128 * j))
            for cp2 in cps:
                cp2.wait()
            compact(tau_vec, splat_i(CAP))

            pltpu.sync_copy(cval.at[pl.ds(0, K)],
                            kval_hbm.at[pl.ds(row * K, K)])
            for i in range(L):
                gidx[pl.ds(i * K, 16)] = cidx[pl.ds(0, 16)] + splat_i(i * NE)
                gidx[pl.ds(i * K + 16, 16)] = (cidx[pl.ds(16, 16)]
                                               + splat_i(i * NE))
            cpl = [pltpu.async_copy(elab_hbm.at[gidx.at[pl.ds(i * K, K)]],
                                    glab.at[pl.ds(i * K, K)], semO)
                   for i in range(L)]
            for cp3 in cpl:
                cp3.wait()
            pltpu.sync_copy(glab, klab_hbm.at[pl.ds(row * L * K, L * K)])
            return 0

        pltpu.sync_copy(tau_hbm.at[pl.ds(row0 * 16, RW * 16)], tbuf)
        lax.fori_loop(0, RW, process_row, 0)

    return sc_kernel(scores_flat, gm_flat, tau_flat, elab_flat)


def _combine_body(kv_ref, kl_ref, le_ref, o_ref):
    kv = kv_ref[...]
    w = jnp.sqrt(kv)
    w = w - jnp.max(w, axis=-1, keepdims=True)
    w = jnp.exp(w)
    w = w / jnp.sum(w, axis=-1, keepdims=True)
    kl = kl_ref[...]
    cls_iota = jax.lax.broadcasted_iota(jnp.int32, (B, K, NC), 2)
    ws = []
    for i in range(L):
        eq = (kl[:, i, :][:, :, None] == cls_iota).astype(jnp.float32)
        ws.append(jnp.sum(eq * w[:, :, None], axis=1))
    W = jnp.concatenate(ws, axis=-1)
    o_ref[...] = jnp.dot(W, le_ref[...], preferred_element_type=jnp.float32)


def _combine(kvalue, klabel_bkl, label_flat):
    return pl.pallas_call(
        _combine_body,
        out_shape=jax.ShapeDtypeStruct((B, D), jnp.float32),
    )(kvalue, klabel_bkl, label_flat)


def kernel(graph_feature, evidence_emb, emb_label, label_emb):
    evidence_pad = jnp.pad(evidence_emb, ((0, NE_PAD - NE), (0, 0)))
    scores, gm = _scores(graph_feature, evidence_pad)
    tau0 = _tau0(gm)
    scores_flat = scores.reshape(B * NE_PAD)
    gm_flat = gm.reshape(B * NGM)
    tau_flat = tau0.reshape(B * 16)
    elab_flat = emb_label.reshape(L * NE).astype(jnp.int32)
    kval_f, _, klab_f = _sc_topk(scores_flat, gm_flat, tau_flat, elab_flat)
    kvalue = kval_f.reshape(B, K)
    klabel = klab_f.reshape(B, L, K)
    label_flat = label_emb.reshape(L * NC, D)
    return _combine(kvalue, klabel, label_flat)
